# Initial kernel scaffold; baseline (speedup 1.0000x reference)
import jax
import jax.numpy as jnp
from jax.experimental import pallas as pl


def kernel(x_t, probs, emb_table, omega_s, omega_a, omega_b):
    raise NotImplementedError("write your pallas kernel here")


# speedup vs baseline: 10.3522x; 10.3522x over previous
import jax
import jax.numpy as jnp
from jax import lax
from jax.experimental import pallas as pl
from jax.experimental.pallas import tpu as pltpu
from jax.experimental.pallas import tpu_sc as plsc

VOCAB = 100000
HIDDEN = 64
BATCH = 32
SEQ = 16
K = 8
MASK_TOKEN_ID = 103

ROWS = BATCH * SEQ
CHUNK = 20000
NCHUNK = VOCAB // CHUNK
VECS = CHUNK // 16

LN2 = 0.6931471805599453
_C5 = 0.043929099810201704
_C4 = -0.18983442828196562
_C3 = 0.4115641479248468
_C2 = -0.7072548989690077
_C1 = 1.4415923923106577
_C0 = 1.4372503465891318e-05

_NC = 2
_NS = 16
_NW = _NC * _NS
_RPW = ROWS // _NW


def _lane_sum(x, lane):
    for sh in (8, 4, 2, 1):
        x = x + x.at[lane ^ sh].get(mode="promise_in_bounds")
    return x


def _lane_max(x, lane):
    for sh in (8, 4, 2, 1):
        x = jnp.maximum(x, x.at[lane ^ sh].get(mode="promise_in_bounds"))
    return x


def _shuf(x, perm):
    return x.at[perm].get(mode="promise_in_bounds")


def _cmpex(keys, vals, lane, j, want_min):
    pk = _shuf(keys, lane ^ j)
    pv = _shuf(vals, lane ^ j)
    nk = jnp.where(want_min, jnp.minimum(keys, pk), jnp.maximum(keys, pk))
    nv = jnp.where(want_min, jnp.where(keys <= pk, vals, pv),
                   jnp.where(keys >= pk, vals, pv))
    return nk, nv


def _bitonic_sort_desc(keys, vals, lane):
    for kb in (1, 2, 3):
        for jb in range(kb - 1, -1, -1):
            want_min = (((lane >> kb) ^ (lane >> jb)) & 1) == 0
            keys, vals = _cmpex(keys, vals, lane, 1 << jb, want_min)
    for jb in (3, 2, 1, 0):
        want_min = ((lane >> jb) & 1) == 1
        keys, vals = _cmpex(keys, vals, lane, 1 << jb, want_min)
    return keys, vals


def _bitonic_merge_desc(keys, vals, lane):
    for jb in (3, 2, 1, 0):
        want_min = ((lane >> jb) & 1) == 1
        keys, vals = _cmpex(keys, vals, lane, 1 << jb, want_min)
    return keys, vals


def _body(probs_hbm, x_hbm, emb_hbm, params_hbm, out_hbm,
          chunk_v, t_val, t_idx, x_v, params_v, idx_v, rows_v, out_v,
          w_v, thr_s, sem):
    wid = lax.axis_index("s") * _NC + lax.axis_index("c")
    lane = lax.iota(jnp.int32, 16)
    neg_inf = jnp.full((16,), -jnp.inf, jnp.float32)

    pltpu.sync_copy(x_hbm.at[pl.ds(pl.multiple_of(wid * _RPW, 8), _RPW)], x_v)
    pltpu.sync_copy(params_hbm, params_v)

    def row_body(j, carry):
        row = wid * _RPW + j
        t_val[...] = neg_inf
        t_idx[...] = jnp.zeros((16,), jnp.int32)
        thr_s[0] = -jnp.inf

        def chunk_body(c, acc):
            off = row * VOCAB + c * CHUNK
            pltpu.sync_copy(
                probs_hbm.at[pl.ds(pl.multiple_of(off, 8), CHUNK)], chunk_v)

            def vec_body(i, acc):
                v = chunk_v[pl.ds(i * 16, 16)]
                bits = lax.bitcast_convert_type(v, jnp.int32)
                e = (bits >> 23) - 127
                m = lax.bitcast_convert_type(
                    (bits & 0x7FFFFF) | 0x3F800000, jnp.float32)
                t = m - 1.0
                poly = _C5 * t + _C4
                poly = poly * t + _C3
                poly = poly * t + _C2
                poly = poly * t + _C1
                poly = poly * t + _C0
                l2 = e.astype(jnp.float32) + poly
                acc = acc + v * l2

                hit = _lane_max(v, lane)[0] > thr_s[0]

                @pl.when(hit)
                def _():
                    ci = c * CHUNK + i * 16 + lane
                    sv, si = _bitonic_sort_desc(v, ci, lane)
                    rv = _shuf(sv, 15 - lane)
                    ri = _shuf(si, 15 - lane)
                    l8 = lane < 8
                    cv = jnp.where(l8, t_val[...], rv)
                    cidx = jnp.where(l8, t_idx[...], ri)
                    mv, mi = _bitonic_merge_desc(cv, cidx, lane)
                    t_val[...] = jnp.where(l8, mv, neg_inf)
                    t_idx[...] = jnp.where(l8, mi, 0)
                    thr_s[0] = mv[7]

                return acc

            return lax.fori_loop(0, VECS, vec_body, acc)

        acc = lax.fori_loop(0, NCHUNK, chunk_body, jnp.zeros((16,), jnp.float32))

        tv = t_val[...]
        ti = t_idx[...]
        xt_vec = x_v[...].at[jnp.full((16,), j, jnp.int32)].get(
            mode="promise_in_bounds")
        l8 = lane < 8
        gidx = jnp.where(l8, ti, 0)
        gidx = jnp.where(lane == 8, xt_vec, gidx)
        idx_v[...] = gidx >> 1
        hbits = gidx & 1
        cp = pltpu.make_async_copy(emb_hbm.at[idx_v], rows_v, sem)
        cp.start()
        cp.wait()

        wr = jnp.where(l8, tv, 0.0)
        w_v[...] = wr / (_lane_sum(wr, lane) + 1e-10)

        pvs = params_v[pl.ds(0, 16)]
        pva = params_v[pl.ds(16, 16)]
        pvb = params_v[pl.ds(32, 16)]
        entropy = -LN2 * _lane_sum(acc, lane)
        inner = pva * (-entropy - pvb)
        lam = pvs / (1.0 + jnp.exp(-inner))
        d = jnp.abs(xt_vec - MASK_TOKEN_ID)
        is_mask_f = (1 - jnp.minimum(d, 1)).astype(jnp.float32)
        lam_eff = lam * is_mask_f

        wv = w_v[...]
        hbits_f = hbits.astype(jnp.float32)
        half = [_shuf(hbits_f, jnp.full((16,), k, jnp.int32))
                for k in range(K + 1)]
        for h in range(HIDDEN // 16):
            def pick(k):
                lo = rows_v[k, pl.ds(h * 16, 16)]
                hi = rows_v[k, pl.ds(64 + h * 16, 16)]
                return lo + half[k] * (hi - lo)
            real = pick(K)
            fb = jnp.zeros((16,), jnp.float32)
            for k in range(K):
                fb = fb + wv[k] * pick(k)
            out_v[pl.ds(h * 16, 16)] = real + lam_eff * (fb - real)

        pltpu.sync_copy(
            out_v, out_hbm.at[pl.ds(pl.multiple_of(row * HIDDEN, 8), HIDDEN)])
        return carry

    lax.fori_loop(0, _RPW, row_body, 0)


@jax.jit
def _run(probs_flat, x_flat, emb_table, params):
    mesh = plsc.VectorSubcoreMesh(core_axis_name="c", subcore_axis_name="s")
    f = pl.kernel(
        _body,
        out_type=jax.ShapeDtypeStruct((ROWS * HIDDEN,), jnp.float32),
        mesh=mesh,
        scratch_types=[
            pltpu.VMEM((CHUNK,), jnp.float32),
            pltpu.VMEM((16,), jnp.float32),
            pltpu.VMEM((16,), jnp.int32),
            pltpu.VMEM((_RPW,), jnp.int32),
            pltpu.VMEM((48,), jnp.float32),
            pltpu.VMEM((16,), jnp.int32),
            pltpu.VMEM((16, 128), jnp.float32),
            pltpu.VMEM((HIDDEN,), jnp.float32),
            pltpu.VMEM((16,), jnp.float32),
            pltpu.SMEM((1,), jnp.float32),
            pltpu.SemaphoreType.DMA,
        ],
    )
    return f(probs_flat, x_flat, emb_table.reshape(VOCAB // 2, 128), params)


def kernel(x_t, probs, emb_table, omega_s, omega_a, omega_b):
    probs_flat = probs.reshape(ROWS * VOCAB)
    x_flat = x_t.reshape(ROWS).astype(jnp.int32)
    params = jnp.concatenate([
        jnp.full((16,), jax.nn.sigmoid(omega_s), jnp.float32),
        jnp.full((16,), jax.nn.softplus(omega_a), jnp.float32),
        jnp.full((16,), -jax.nn.softplus(omega_b), jnp.float32),
    ])
    out = _run(probs_flat, x_flat, emb_table, params)
    return out.reshape(BATCH, SEQ, HIDDEN)
